# baseline (speedup 1.0000x reference)
import functools

import jax
import jax.numpy as jnp
from jax import lax
from jax.experimental import pallas as pl
from jax.experimental.pallas import tpu as pltpu
from jax.experimental.pallas import tpu_sc as plsc

N_SRC = 64
C = 3
P = 128
H = 2048
W = 2048
SY = 8
NC = 2
NS = 16
NW = NC * NS
ROUNDS = H // (SY * NW)
STRIDE = SY * NW
WIN = P + SY - 1
HALF = C * SY * W
SHALF = C * SY * P


def _sc_scatter(patch_flat, ys, xs):
    mesh = plsc.VectorSubcoreMesh(core_axis_name="c", subcore_axis_name="s")

    @functools.partial(
        pl.kernel,
        out_type=jax.ShapeDtypeStruct((C, H, W), jnp.float32),
        mesh=mesh,
        scratch_types=[
            pltpu.VMEM((2 * HALF,), jnp.float32),
            pltpu.VMEM((2 * SHALF,), jnp.float32),
            pltpu.VMEM((N_SRC + 16,), jnp.int32),
            pltpu.VMEM((N_SRC + 16,), jnp.int32),
            pltpu.VMEM((ROUNDS * N_SRC * 16,), jnp.int32),
            pltpu.VMEM((ROUNDS * 16,), jnp.int32),
            pltpu.SemaphoreType.DMA,
            pltpu.SemaphoreType.DMA,
            pltpu.SemaphoreType.DMA,
            pltpu.SemaphoreType.DMA,
            pltpu.SemaphoreType.DMA,
            pltpu.SemaphoreType.DMA,
            pltpu.SemaphoreType.DMA,
            pltpu.SemaphoreType.DMA,
            pltpu.SemaphoreType.DMA,
            pltpu.SemaphoreType.DMA,
        ],
    )
    def body(patch_hbm, ys_hbm, xs_hbm, zeros_hbm, out_hbm, slab, stage,
             ys_v, xs_v, list_v, cnt_v, f00, f01, f02, f10, f11, f12,
             semw0, semw1, semz0, semz1):
        fsems = ((f00, f01, f02), (f10, f11, f12))
        semws = (semw0, semw1)
        semzs = (semz0, semz1)
        wid = lax.axis_index("s") * NC + lax.axis_index("c")
        pltpu.sync_copy(ys_hbm, ys_v)
        pltpu.sync_copy(xs_hbm, xs_v)
        lanes = lax.broadcasted_iota(jnp.int32, (16,), 0)
        zi16 = jnp.zeros((16,), jnp.int32)

        for r in range(ROUNDS):
            cnt_v[pl.ds(r * 16, 16)] = zi16

        def scan_body(i, _):
            y = ys_v[pl.ds(i, 16)][0]
            u = y - wid * SY + (P - 1)

            @pl.when(jnp.logical_and(u >= 0, u % STRIDE < WIN))
            def _():
                r = u // STRIDE
                n = cnt_v[pl.ds(r * 16, 16)][0]
                list_v[pl.ds((r * N_SRC + n) * 16, 16)] = lanes * 0 + i
                cnt_v[pl.ds(r * 16, 16)] = lanes * 0 + (n + 1)

            return 0

        with jax.named_scope("scan"):
            lax.fori_loop(0, N_SRC, scan_body, 0)

        def fire(r, j, slot):
            i = list_v[pl.ds((r * N_SRC + j) * 16, 16)][0]
            y = ys_v[pl.ds(i, 16)][0]
            dy = wid * SY + r * STRIDE - y
            fs = jnp.clip(dy, 0, P - SY)
            for c in range(C):
                src = patch_hbm.at[pl.ds(((i * C + c) * P + fs) * P, SY * P)]
                dst = stage.at[pl.ds(slot * SHALF + c * SY * P, SY * P)]
                pltpu.async_copy(src, dst, fsems[slot][c])

        def accumulate(r, j, slot, base):
            i = list_v[pl.ds((r * N_SRC + j) * 16, 16)][0]
            y = ys_v[pl.ds(i, 16)][0]
            x = xs_v[pl.ds(i, 16)][0]
            dy = wid * SY + r * STRIDE - y
            fs = jnp.clip(dy, 0, P - SY)
            for c in range(C):
                pltpu.make_async_copy(
                    patch_hbm.at[pl.ds(0, SY * P)],
                    stage.at[pl.ds(slot * SHALF, SY * P)],
                    fsems[slot][c],
                ).wait()

            ra = jnp.maximum(0, -dy)
            rb = jnp.minimum(SY, P - dy)
            for c in range(C):
                def row_body(rr, _):
                    srow = rr + dy - fs
                    sbase = slot * SHALF + c * SY * P + srow * P
                    dbase = base + (c * SY + rr) * W + x
                    for u in range(P // 16):
                        v = stage[pl.ds(sbase + u * 16, 16)]
                        plsc.addupdate(
                            slab.at[pl.ds(dbase + u * 16, 16)], v
                        )
                    return 0

                lax.fori_loop(ra, rb, row_body, 0)

        def wb_wait_all(parity):
            def wb_wait(j, _):
                pltpu.make_async_copy(
                    slab.at[pl.ds(0, W)], out_hbm.at[0, 0, :], semws[parity]
                ).wait()
                return 0

            lax.fori_loop(0, C * SY, wb_wait, 0)

        def fire_zero(parity):
            pltpu.async_copy(
                zeros_hbm, slab.at[pl.ds(parity * HALF, HALF)], semzs[parity]
            )

        def run_round(r, rp, parity):
            base = parity * HALF
            y0 = wid * SY + r * STRIDE
            n_r = cnt_v[pl.ds(r * 16, 16)][0]

            @pl.when(n_r > 0)
            def _():
                fire(r, 0, 0)

            with jax.named_scope("zwait"):
                pltpu.make_async_copy(
                    zeros_hbm, slab.at[pl.ds(base, HALF)], semzs[parity]
                ).wait()

            def pair_body(t, _):
                j = 2 * t

                @pl.when(j + 1 < n_r)
                def _():
                    fire(r, j + 1, 1)

                accumulate(r, j, 0, base)

                @pl.when(j + 2 < n_r)
                def _():
                    fire(r, j + 2, 0)

                @pl.when(j + 1 < n_r)
                def _():
                    accumulate(r, j + 1, 1, base)

                return 0

            with jax.named_scope("srcs"):
                lax.fori_loop(0, (n_r + 1) // 2, pair_body, 0)

            def wb_body(j, _):
                c = j // SY
                rr = j % SY
                src = slab.at[pl.ds(base + (c * SY + rr) * W, W)]
                dst = out_hbm.at[c, y0 + rr, :]
                pltpu.async_copy(src, dst, semws[parity])
                return 0

            lax.fori_loop(0, C * SY, wb_body, 0)

            with jax.named_scope("prep"):
                if parity == 0:
                    @pl.when(rp >= 1)
                    def _():
                        wb_wait_all(1)
                        fire_zero(1)
                else:
                    @pl.when(rp <= ROUNDS // 2 - 2)
                    def _():
                        wb_wait_all(0)
                        fire_zero(0)

        fire_zero(0)
        fire_zero(1)

        def round_pair(rp, _):
            run_round(2 * rp, rp, 0)
            run_round(2 * rp + 1, rp, 1)
            return 0

        lax.fori_loop(0, ROUNDS // 2, round_pair, 0)

        for p in range(2):
            def wb_wait_final(j, _):
                pltpu.make_async_copy(
                    slab.at[pl.ds(0, W)], out_hbm.at[0, 0, :], semws[p]
                ).wait()
                return 0

            lax.fori_loop(0, C * SY, wb_wait_final, 0)

    return body(patch_flat, ys, xs, jnp.zeros((HALF,), jnp.float32))


def kernel(source_models, origins):
    patch_flat = source_models.reshape(-1)
    origins = origins.astype(jnp.int32)
    ys = jnp.pad(origins[:, 0], (0, 16))
    xs = jnp.pad(origins[:, 1], (0, 16))
    return _sc_scatter(patch_flat, ys, xs)


# speedup vs baseline: 1.3637x; 1.3637x over previous
import functools

import jax
import jax.numpy as jnp
from jax import lax
from jax.experimental import pallas as pl
from jax.experimental.pallas import tpu as pltpu
from jax.experimental.pallas import tpu_sc as plsc

N_SRC = 64
C = 3
P = 128
H = 2048
W = 2048
SY = 8
NC = 2
NS = 16
NW = NC * NS
ROUNDS = H // (SY * NW)
STRIDE = SY * NW
WIN = P + SY - 1
HALF = C * SY * W
SHALF = C * SY * P


def _sc_scatter(patch_flat, ys, xs):
    mesh = plsc.VectorSubcoreMesh(core_axis_name="c", subcore_axis_name="s")

    @functools.partial(
        pl.kernel,
        out_type=jax.ShapeDtypeStruct((C, H, W), jnp.float32),
        mesh=mesh,
        scratch_types=[
            pltpu.VMEM((2 * HALF,), jnp.float32),
            pltpu.VMEM((2 * SHALF,), jnp.float32),
            pltpu.VMEM_SHARED((HALF,), jnp.float32),
            pltpu.VMEM((N_SRC + 16,), jnp.int32),
            pltpu.VMEM((N_SRC + 16,), jnp.int32),
            pltpu.VMEM((ROUNDS * N_SRC * 16,), jnp.int32),
            pltpu.VMEM((ROUNDS * 16,), jnp.int32),
            pltpu.SemaphoreType.DMA,
            pltpu.SemaphoreType.DMA,
            pltpu.SemaphoreType.DMA,
            pltpu.SemaphoreType.DMA,
            pltpu.SemaphoreType.DMA,
            pltpu.SemaphoreType.DMA,
            pltpu.SemaphoreType.DMA,
            pltpu.SemaphoreType.DMA,
            pltpu.SemaphoreType.DMA,
            pltpu.SemaphoreType.DMA,
        ],
    )
    def body(patch_hbm, ys_hbm, xs_hbm, out_hbm, slab, stage, zshared,
             ys_v, xs_v, list_v, cnt_v, f00, f01, f02, f10, f11, f12,
             semw0, semw1, semz0, semz1):
        fsems = ((f00, f01, f02), (f10, f11, f12))
        semws = (semw0, semw1)
        semzs = (semz0, semz1)
        wid = lax.axis_index("s") * NC + lax.axis_index("c")
        pltpu.sync_copy(ys_hbm, ys_v)
        pltpu.sync_copy(xs_hbm, xs_v)
        lanes = lax.broadcasted_iota(jnp.int32, (16,), 0)
        zi16 = jnp.zeros((16,), jnp.int32)
        zeros16 = jnp.zeros((16,), jnp.float32)

        def zinit_body(j, _):
            for u in range(16):
                slab[pl.ds(j * 256 + u * 16, 16)] = zeros16
            return 0

        lax.fori_loop(0, 2 * HALF // 256, zinit_body, 0)

        @pl.when(lax.axis_index("s") == 0)
        def _():
            pltpu.sync_copy(slab.at[pl.ds(0, HALF)], zshared)

        plsc.subcore_barrier()

        for r in range(ROUNDS):
            cnt_v[pl.ds(r * 16, 16)] = zi16

        def scan_body(i, _):
            y = ys_v[pl.ds(i, 16)][0]
            u = y - wid * SY + (P - 1)

            @pl.when(jnp.logical_and(u >= 0, u % STRIDE < WIN))
            def _():
                r = u // STRIDE
                n = cnt_v[pl.ds(r * 16, 16)][0]
                list_v[pl.ds((r * N_SRC + n) * 16, 16)] = lanes * 0 + i
                cnt_v[pl.ds(r * 16, 16)] = lanes * 0 + (n + 1)

            return 0

        with jax.named_scope("scan"):
            lax.fori_loop(0, N_SRC, scan_body, 0)

        def fire(r, j, slot):
            i = list_v[pl.ds((r * N_SRC + j) * 16, 16)][0]
            y = ys_v[pl.ds(i, 16)][0]
            dy = wid * SY + r * STRIDE - y
            fs = jnp.clip(dy, 0, P - SY)
            for c in range(C):
                src = patch_hbm.at[pl.ds(((i * C + c) * P + fs) * P, SY * P)]
                dst = stage.at[pl.ds(slot * SHALF + c * SY * P, SY * P)]
                pltpu.async_copy(src, dst, fsems[slot][c])

        def accumulate(r, j, slot, base):
            i = list_v[pl.ds((r * N_SRC + j) * 16, 16)][0]
            y = ys_v[pl.ds(i, 16)][0]
            x = xs_v[pl.ds(i, 16)][0]
            dy = wid * SY + r * STRIDE - y
            fs = jnp.clip(dy, 0, P - SY)
            for c in range(C):
                pltpu.make_async_copy(
                    patch_hbm.at[pl.ds(0, SY * P)],
                    stage.at[pl.ds(slot * SHALF, SY * P)],
                    fsems[slot][c],
                ).wait()

            ra = jnp.maximum(0, -dy)
            rb = jnp.minimum(SY, P - dy)
            for c in range(C):
                def row_body(rr, _):
                    srow = rr + dy - fs
                    sbase = slot * SHALF + c * SY * P + srow * P
                    dbase = base + (c * SY + rr) * W + x
                    for u in range(P // 16):
                        v = stage[pl.ds(sbase + u * 16, 16)]
                        plsc.addupdate(
                            slab.at[pl.ds(dbase + u * 16, 16)], v
                        )
                    return 0

                lax.fori_loop(ra, rb, row_body, 0)

        def wb_wait_all(parity):
            def wb_wait(j, _):
                pltpu.make_async_copy(
                    slab.at[pl.ds(0, W)], out_hbm.at[0, 0, :], semws[parity]
                ).wait()
                return 0

            lax.fori_loop(0, C * SY, wb_wait, 0)

        def fire_zero(parity):
            pltpu.async_copy(
                zshared, slab.at[pl.ds(parity * HALF, HALF)], semzs[parity]
            )

        def run_round(r, rp, parity):
            base = parity * HALF
            y0 = wid * SY + r * STRIDE
            n_r = cnt_v[pl.ds(r * 16, 16)][0]

            @pl.when(n_r > 0)
            def _():
                fire(r, 0, 0)

            with jax.named_scope("zwait"):
                @pl.when(rp >= 1)
                def _():
                    pltpu.make_async_copy(
                        zshared, slab.at[pl.ds(base, HALF)], semzs[parity]
                    ).wait()

            def pair_body(t, _):
                j = 2 * t

                @pl.when(j + 1 < n_r)
                def _():
                    fire(r, j + 1, 1)

                accumulate(r, j, 0, base)

                @pl.when(j + 2 < n_r)
                def _():
                    fire(r, j + 2, 0)

                @pl.when(j + 1 < n_r)
                def _():
                    accumulate(r, j + 1, 1, base)

                return 0

            with jax.named_scope("srcs"):
                lax.fori_loop(0, (n_r + 1) // 2, pair_body, 0)

            def wb_body(j, _):
                c = j // SY
                rr = j % SY
                src = slab.at[pl.ds(base + (c * SY + rr) * W, W)]
                dst = out_hbm.at[c, y0 + rr, :]
                pltpu.async_copy(src, dst, semws[parity])
                return 0

            lax.fori_loop(0, C * SY, wb_body, 0)

            with jax.named_scope("prep"):
                if parity == 0:
                    @pl.when(rp >= 1)
                    def _():
                        wb_wait_all(1)
                        fire_zero(1)
                else:
                    @pl.when(rp <= ROUNDS // 2 - 2)
                    def _():
                        wb_wait_all(0)
                        fire_zero(0)

        def round_pair(rp, _):
            run_round(2 * rp, rp, 0)
            run_round(2 * rp + 1, rp, 1)
            return 0

        lax.fori_loop(0, ROUNDS // 2, round_pair, 0)

        for p in range(2):
            def wb_wait_final(j, _):
                pltpu.make_async_copy(
                    slab.at[pl.ds(0, W)], out_hbm.at[0, 0, :], semws[p]
                ).wait()
                return 0

            lax.fori_loop(0, C * SY, wb_wait_final, 0)

    return body(patch_flat, ys, xs)


def kernel(source_models, origins):
    patch_flat = source_models.reshape(-1)
    origins = origins.astype(jnp.int32)
    ys = jnp.pad(origins[:, 0], (0, 16))
    xs = jnp.pad(origins[:, 1], (0, 16))
    return _sc_scatter(patch_flat, ys, xs)


# speedup vs baseline: 1.4426x; 1.0578x over previous
import functools

import jax
import jax.numpy as jnp
from jax import lax
from jax.experimental import pallas as pl
from jax.experimental.pallas import tpu as pltpu
from jax.experimental.pallas import tpu_sc as plsc

N_SRC = 64
C = 3
P = 128
H = 2048
W = 2048
SY = 8
NC = 2
NS = 16
NW = NC * NS
ROUNDS = H // (SY * NW)
STRIDE = SY * NW
WIN = P + SY - 1
HALF = C * SY * W
SHALF = C * SY * P


def _sc_scatter(patch_flat, ys, xs):
    mesh = plsc.VectorSubcoreMesh(core_axis_name="c", subcore_axis_name="s")

    @functools.partial(
        pl.kernel,
        out_type=jax.ShapeDtypeStruct((C, H, W), jnp.float32),
        mesh=mesh,
        scratch_types=[
            pltpu.VMEM((2 * HALF,), jnp.float32),
            pltpu.VMEM((2 * SHALF,), jnp.float32),
            pltpu.VMEM((N_SRC + 16,), jnp.int32),
            pltpu.VMEM((N_SRC + 16,), jnp.int32),
            pltpu.VMEM((ROUNDS * N_SRC * 16,), jnp.int32),
            pltpu.VMEM((ROUNDS * 16,), jnp.int32),
            pltpu.SemaphoreType.DMA,
            pltpu.SemaphoreType.DMA,
            pltpu.SemaphoreType.DMA,
            pltpu.SemaphoreType.DMA,
            pltpu.SemaphoreType.DMA,
            pltpu.SemaphoreType.DMA,
            pltpu.SemaphoreType.DMA,
            pltpu.SemaphoreType.DMA,
        ],
    )
    def body(patch_hbm, ys_hbm, xs_hbm, out_hbm, slab, stage,
             ys_v, xs_v, list_v, cnt_v, f00, f01, f02, f10, f11, f12,
             semw0, semw1):
        fsems = ((f00, f01, f02), (f10, f11, f12))
        semws = (semw0, semw1)
        wid = lax.axis_index("s") * NC + lax.axis_index("c")
        pltpu.sync_copy(ys_hbm, ys_v)
        pltpu.sync_copy(xs_hbm, xs_v)
        lanes = lax.broadcasted_iota(jnp.int32, (16,), 0)
        zi16 = jnp.zeros((16,), jnp.int32)
        zeros16 = jnp.zeros((16,), jnp.float32)

        for r in range(ROUNDS):
            cnt_v[pl.ds(r * 16, 16)] = zi16

        def scan_body(i, _):
            y = ys_v[pl.ds(i, 16)][0]
            u = y - wid * SY + (P - 1)

            @pl.when(jnp.logical_and(u >= 0, u % STRIDE < WIN))
            def _():
                r = u // STRIDE
                n = cnt_v[pl.ds(r * 16, 16)][0]
                list_v[pl.ds((r * N_SRC + n) * 16, 16)] = lanes * 0 + i
                cnt_v[pl.ds(r * 16, 16)] = lanes * 0 + (n + 1)

            return 0

        lax.fori_loop(0, N_SRC, scan_body, 0)

        def fire(r, j, slot):
            i = list_v[pl.ds((r * N_SRC + j) * 16, 16)][0]
            y = ys_v[pl.ds(i, 16)][0]
            dy = wid * SY + r * STRIDE - y
            fs = jnp.clip(dy, 0, P - SY)
            for c in range(C):
                src = patch_hbm.at[pl.ds(((i * C + c) * P + fs) * P, SY * P)]
                dst = stage.at[pl.ds(slot * SHALF + c * SY * P, SY * P)]
                pltpu.async_copy(src, dst, fsems[slot][c])

        def accumulate(r, j, slot, base):
            i = list_v[pl.ds((r * N_SRC + j) * 16, 16)][0]
            y = ys_v[pl.ds(i, 16)][0]
            x = xs_v[pl.ds(i, 16)][0]
            dy = wid * SY + r * STRIDE - y
            fs = jnp.clip(dy, 0, P - SY)
            for c in range(C):
                pltpu.make_async_copy(
                    patch_hbm.at[pl.ds(0, SY * P)],
                    stage.at[pl.ds(slot * SHALF, SY * P)],
                    fsems[slot][c],
                ).wait()

            ra = jnp.maximum(0, -dy)
            rb = jnp.minimum(SY, P - dy)
            for c in range(C):
                def row_body(rr, _):
                    srow = rr + dy - fs
                    sbase = slot * SHALF + c * SY * P + srow * P
                    dbase = base + (c * SY + rr) * W + x
                    for u in range(P // 16):
                        v = stage[pl.ds(sbase + u * 16, 16)]
                        plsc.addupdate(
                            slab.at[pl.ds(dbase + u * 16, 16)], v
                        )
                    return 0

                lax.fori_loop(ra, rb, row_body, 0)

        def wb_wait_all(parity):
            def wb_wait(j, _):
                pltpu.make_async_copy(
                    slab.at[pl.ds(0, W)], out_hbm.at[0, 0, :], semws[parity]
                ).wait()
                return 0

            lax.fori_loop(0, C * SY, wb_wait, 0)

        def run_round(r, rp, parity):
            base = parity * HALF
            y0 = wid * SY + r * STRIDE
            n_r = cnt_v[pl.ds(r * 16, 16)][0]

            @pl.when(n_r > 0)
            def _():
                fire(r, 0, 0)

            @pl.when(rp >= 1)
            def _():
                wb_wait_all(parity)

            def zero_body(j, _):
                for u in range(16):
                    slab[pl.ds(base + j * 256 + u * 16, 16)] = zeros16
                return 0

            lax.fori_loop(0, HALF // 256, zero_body, 0)

            def pair_body(t, _):
                j = 2 * t

                @pl.when(j + 1 < n_r)
                def _():
                    fire(r, j + 1, 1)

                accumulate(r, j, 0, base)

                @pl.when(j + 2 < n_r)
                def _():
                    fire(r, j + 2, 0)

                @pl.when(j + 1 < n_r)
                def _():
                    accumulate(r, j + 1, 1, base)

                return 0

            lax.fori_loop(0, (n_r + 1) // 2, pair_body, 0)

            def wb_body(j, _):
                c = j // SY
                rr = j % SY
                src = slab.at[pl.ds(base + (c * SY + rr) * W, W)]
                dst = out_hbm.at[c, y0 + rr, :]
                pltpu.async_copy(src, dst, semws[parity])
                return 0

            lax.fori_loop(0, C * SY, wb_body, 0)

        def round_pair(rp, _):
            run_round(2 * rp, rp, 0)
            run_round(2 * rp + 1, rp, 1)
            return 0

        lax.fori_loop(0, ROUNDS // 2, round_pair, 0)

        for p in range(2):
            def wb_wait_final(j, _):
                pltpu.make_async_copy(
                    slab.at[pl.ds(0, W)], out_hbm.at[0, 0, :], semws[p]
                ).wait()
                return 0

            lax.fori_loop(0, C * SY, wb_wait_final, 0)

    return body(patch_flat, ys, xs)


def kernel(source_models, origins):
    patch_flat = source_models.reshape(-1)
    origins = origins.astype(jnp.int32)
    ys = jnp.pad(origins[:, 0], (0, 16))
    xs = jnp.pad(origins[:, 1], (0, 16))
    return _sc_scatter(patch_flat, ys, xs)


# speedup vs baseline: 1.4600x; 1.0121x over previous
import functools

import jax
import jax.numpy as jnp
from jax import lax
from jax.experimental import pallas as pl
from jax.experimental.pallas import tpu as pltpu
from jax.experimental.pallas import tpu_sc as plsc

N_SRC = 64
C = 3
P = 128
H = 2048
W = 2048
SY = 8
NC = 2
NS = 16
NW = NC * NS
ROUNDS = H // (SY * NW)
STRIDE = SY * NW
WIN = P + SY - 1
HALF = C * SY * W
SHALF = C * SY * P


def _sc_scatter(patch_flat, ys, xs):
    mesh = plsc.VectorSubcoreMesh(core_axis_name="c", subcore_axis_name="s")

    @functools.partial(
        pl.kernel,
        out_type=jax.ShapeDtypeStruct((C, H, W), jnp.float32),
        mesh=mesh,
        scratch_types=[
            pltpu.VMEM((2 * HALF,), jnp.float32),
            pltpu.VMEM((3 * SHALF,), jnp.float32),
            pltpu.VMEM((N_SRC + 16,), jnp.int32),
            pltpu.VMEM((N_SRC + 16,), jnp.int32),
            pltpu.VMEM((ROUNDS * N_SRC * 16,), jnp.int32),
            pltpu.VMEM((ROUNDS * 16,), jnp.int32),
            pltpu.SemaphoreType.DMA,
            pltpu.SemaphoreType.DMA,
            pltpu.SemaphoreType.DMA,
            pltpu.SemaphoreType.DMA,
            pltpu.SemaphoreType.DMA,
            pltpu.SemaphoreType.DMA,
            pltpu.SemaphoreType.DMA,
            pltpu.SemaphoreType.DMA,
            pltpu.SemaphoreType.DMA,
            pltpu.SemaphoreType.DMA,
            pltpu.SemaphoreType.DMA,
        ],
    )
    def body(patch_hbm, ys_hbm, xs_hbm, out_hbm, slab, stage,
             ys_v, xs_v, list_v, cnt_v, f00, f01, f02, f10, f11, f12,
             f20, f21, f22, semw0, semw1):
        fsems = ((f00, f01, f02), (f10, f11, f12), (f20, f21, f22))
        semws = (semw0, semw1)
        wid = lax.axis_index("s") * NC + lax.axis_index("c")
        pltpu.sync_copy(ys_hbm, ys_v)
        pltpu.sync_copy(xs_hbm, xs_v)
        lanes = lax.broadcasted_iota(jnp.int32, (16,), 0)
        zi16 = jnp.zeros((16,), jnp.int32)
        zeros16 = jnp.zeros((16,), jnp.float32)

        for r in range(ROUNDS):
            cnt_v[pl.ds(r * 16, 16)] = zi16

        def scan_body(i, _):
            y = ys_v[pl.ds(i, 16)][0]
            u = y - wid * SY + (P - 1)

            @pl.when(jnp.logical_and(u >= 0, u % STRIDE < WIN))
            def _():
                r = u // STRIDE
                n = cnt_v[pl.ds(r * 16, 16)][0]
                list_v[pl.ds((r * N_SRC + n) * 16, 16)] = lanes * 0 + i
                cnt_v[pl.ds(r * 16, 16)] = lanes * 0 + (n + 1)

            return 0

        lax.fori_loop(0, N_SRC, scan_body, 0)

        def fire(r, j, slot):
            i = list_v[pl.ds((r * N_SRC + j) * 16, 16)][0]
            y = ys_v[pl.ds(i, 16)][0]
            dy = wid * SY + r * STRIDE - y
            fs = jnp.clip(dy, 0, P - SY)
            for c in range(C):
                src = patch_hbm.at[pl.ds(((i * C + c) * P + fs) * P, SY * P)]
                dst = stage.at[pl.ds(slot * SHALF + c * SY * P, SY * P)]
                pltpu.async_copy(src, dst, fsems[slot][c])

        def accumulate(r, j, slot, base):
            i = list_v[pl.ds((r * N_SRC + j) * 16, 16)][0]
            y = ys_v[pl.ds(i, 16)][0]
            x = xs_v[pl.ds(i, 16)][0]
            dy = wid * SY + r * STRIDE - y
            fs = jnp.clip(dy, 0, P - SY)
            for c in range(C):
                pltpu.make_async_copy(
                    patch_hbm.at[pl.ds(0, SY * P)],
                    stage.at[pl.ds(slot * SHALF, SY * P)],
                    fsems[slot][c],
                ).wait()

            ra = jnp.maximum(0, -dy)
            rb = jnp.minimum(SY, P - dy)
            for c in range(C):
                def row_body(rr, _):
                    srow = rr + dy - fs
                    sbase = slot * SHALF + c * SY * P + srow * P
                    dbase = base + (c * SY + rr) * W + x
                    for u in range(P // 16):
                        v = stage[pl.ds(sbase + u * 16, 16)]
                        plsc.addupdate(
                            slab.at[pl.ds(dbase + u * 16, 16)], v
                        )
                    return 0

                lax.fori_loop(ra, rb, row_body, 0)

        def wb_wait_all(parity):
            def wb_wait(j, _):
                pltpu.make_async_copy(
                    slab.at[pl.ds(0, W)], out_hbm.at[0, 0, :], semws[parity]
                ).wait()
                return 0

            lax.fori_loop(0, C * SY, wb_wait, 0)

        def run_round(r, rp, parity):
            base = parity * HALF
            y0 = wid * SY + r * STRIDE
            n_r = cnt_v[pl.ds(r * 16, 16)][0]

            @pl.when(n_r > 0)
            def _():
                fire(r, 0, 0)

            @pl.when(n_r > 1)
            def _():
                fire(r, 1, 1)

            @pl.when(rp >= 1)
            def _():
                wb_wait_all(parity)

            def zero_body(j, _):
                for u in range(16):
                    slab[pl.ds(base + j * 256 + u * 16, 16)] = zeros16
                return 0

            lax.fori_loop(0, HALF // 256, zero_body, 0)

            def triple_body(t, _):
                j = 3 * t

                @pl.when(j + 2 < n_r)
                def _():
                    fire(r, j + 2, 2)

                accumulate(r, j, 0, base)

                @pl.when(j + 3 < n_r)
                def _():
                    fire(r, j + 3, 0)

                @pl.when(j + 1 < n_r)
                def _():
                    accumulate(r, j + 1, 1, base)

                @pl.when(j + 4 < n_r)
                def _():
                    fire(r, j + 4, 1)

                @pl.when(j + 2 < n_r)
                def _():
                    accumulate(r, j + 2, 2, base)

                return 0

            lax.fori_loop(0, (n_r + 2) // 3, triple_body, 0)

            def wb_body(j, _):
                c = j // SY
                rr = j % SY
                src = slab.at[pl.ds(base + (c * SY + rr) * W, W)]
                dst = out_hbm.at[c, y0 + rr, :]
                pltpu.async_copy(src, dst, semws[parity])
                return 0

            lax.fori_loop(0, C * SY, wb_body, 0)

        def round_pair(rp, _):
            run_round(2 * rp, rp, 0)
            run_round(2 * rp + 1, rp, 1)
            return 0

        lax.fori_loop(0, ROUNDS // 2, round_pair, 0)

        for p in range(2):
            def wb_wait_final(j, _):
                pltpu.make_async_copy(
                    slab.at[pl.ds(0, W)], out_hbm.at[0, 0, :], semws[p]
                ).wait()
                return 0

            lax.fori_loop(0, C * SY, wb_wait_final, 0)

    return body(patch_flat, ys, xs)


def kernel(source_models, origins):
    patch_flat = source_models.reshape(-1)
    origins = origins.astype(jnp.int32)
    ys = jnp.pad(origins[:, 0], (0, 16))
    xs = jnp.pad(origins[:, 1], (0, 16))
    return _sc_scatter(patch_flat, ys, xs)
